# baseline (speedup 1.0000x reference)
import functools

import jax
import jax.numpy as jnp
from jax import lax
from jax.experimental import pallas as pl
from jax.experimental.pallas import tpu as pltpu
from jax.experimental.pallas import tpu_sc as plsc

N = 10000
CIN = 256
CH = 128
E = 160000
NS = 16
NC = 2
ET = E // NS
K = 64
NCH = 160
NCHH = NCH // 2
NBUF = 4
EP = NCH * K
PAD_SPREAD = 128
H = 10240
SLICE = H // NS
L = 16
RB = 40
NRCH = N // RB


def _rsqrt_newton(d):
    i = plsc.bitcast(d, jnp.int32)
    y = plsc.bitcast(jnp.int32(0x5F3759DF) - (i >> 1), jnp.float32)
    for _ in range(3):
        y = y * (1.5 - 0.5 * d * y * y)
    return y


def _sc_aggregate(x3, gsd):
    mesh = plsc.VectorSubcoreMesh(core_axis_name="c", subcore_axis_name="s")

    out_type = [
        jax.ShapeDtypeStruct((N, CH), jnp.float32),
        jax.ShapeDtypeStruct((N, CH), jnp.float32),
        jax.ShapeDtypeStruct((H,), jnp.float32),
        jax.ShapeDtypeStruct((N, CH), jnp.float32),
        jax.ShapeDtypeStruct((N, CH), jnp.float32),
    ]
    scratch = [
        pltpu.VMEM((NCHH, 2, K), jnp.int32),
        pltpu.VMEM((K,), jnp.float32),
        pltpu.VMEM((SLICE,), jnp.float32),
        pltpu.VMEM((K,), jnp.float32),
        pltpu.VMEM((K, CH), jnp.float32),
        pltpu.VMEM((K, CH), jnp.float32),
        pltpu.VMEM((K, CH), jnp.float32),
        pltpu.VMEM((K, CH), jnp.float32),
        pltpu.VMEM_SHARED((H, CH), jnp.float32),
        pltpu.VMEM_SHARED((H,), jnp.float32),
        pltpu.VMEM_SHARED((H,), jnp.float32),
    ] + [pltpu.SemaphoreType.DMA] * (2 * NBUF)

    @functools.partial(
        pl.kernel, out_type=out_type, mesh=mesh, scratch_types=scratch,
        compiler_params=pltpu.CompilerParams(use_tc_tiling_on_sc=False,
                                             needs_layout_passes=False))
    def body(x3_hbm, gsd_hbm, zA_hbm, zB_hbm, dinv_hbm, u0_hbm, u1_hbm,
             sd_t, ones_t, sl_t, dv_t, buf0, buf1, buf2, buf3,
             z_sp, hist_sp, dinv_sp,
             semg0, semg1, semg2, semg3, sems0, sems1, sems2, sems3):
        bufs = (buf0, buf1, buf2, buf3)
        semg = (semg0, semg1, semg2, semg3)
        sems = (sems0, sems1, sems2, sems3)
        c = lax.axis_index("c")
        s = lax.axis_index("s")
        zeros16 = jnp.zeros((L,), jnp.float32)
        ones16 = jnp.ones((L,), jnp.float32)

        def zero_sl(i, _):
            sl_t[pl.ds(i * L, L)] = zeros16
            return 0
        lax.fori_loop(0, SLICE // L, zero_sl, 0, unroll=4)

        def zero_buf0(i, _):
            for k in range(CH // L):
                buf0[i, pl.ds(k * L, L)] = zeros16
            return 0
        lax.fori_loop(0, K, zero_buf0, 0, unroll=2)

        for k in range(K // L):
            ones_t[pl.ds(k * L, L)] = ones16

        pltpu.sync_copy(sl_t, hist_sp.at[pl.ds(s * SLICE, SLICE)])
        plsc.subcore_barrier()

        _p1 = jax.named_scope("p1_degree"); _p1.__enter__()
        D = 16

        def hfire(j):
            pltpu.async_copy(ones_t, hist_sp.at[sd_t.at[j, 1]], semg0,
                             add=True)

        def hdrain(j, _):
            pltpu.make_async_copy(ones_t, hist_sp.at[sd_t.at[0, 1]], semg0
                                  ).wait()
            return 0

        for h in range(2):
            pltpu.sync_copy(gsd_hbm.at[s, pl.ds(h * NCHH, NCHH)], sd_t)
            for j in range(D):
                hfire(j)

            def hist_step(j, _):
                hfire(j + D)
                hdrain(j, None)
                return 0
            lax.fori_loop(0, NCHH - D, hist_step, 0)
            lax.fori_loop(0, D, hdrain, 0)
        plsc.subcore_barrier()

        _p1.__exit__(None, None, None)
        _p2 = jax.named_scope("p2_dinv_zero"); _p2.__enter__()
        pltpu.sync_copy(hist_sp.at[pl.ds(s * SLICE, SLICE)], sl_t)

        def dinv_step(m, _):
            d = sl_t[pl.ds(m * L, L)] + 1.0
            sl_t[pl.ds(m * L, L)] = _rsqrt_newton(d)
            return 0
        lax.fori_loop(0, SLICE // L, dinv_step, 0)
        pltpu.sync_copy(sl_t, dinv_sp.at[pl.ds(s * SLICE, SLICE)])

        for m in range(SLICE // K):
            pltpu.sync_copy(buf0, z_sp.at[pl.ds(s * SLICE + m * K, K)])
        plsc.subcore_barrier()

        _p2.__exit__(None, None, None)
        nu = jnp.where(s < NRCH - (NRCH // NS) * NS, NRCH // NS + 1,
                       NRCH // NS)

        def halfwork(u_hbm, z_hbm, hw_c):
            def u_chunk(kk, _):
                base = (s + kk * NS) * RB
                d = pltpu.async_copy(dinv_sp.at[pl.ds(base, RB)],
                                     dv_t.at[pl.ds(0, RB)], semg1)
                pltpu.sync_copy(x3_hbm.at[pl.ds(base, RB), hw_c],
                                buf1.at[pl.ds(0, RB)])
                d.wait()

                def u_row(i, _):
                    dv = dv_t[pl.ds(i, L)][0]
                    for k in range(CH // L):
                        buf1[i, pl.ds(k * L, L)] = (
                            buf1[i, pl.ds(k * L, L)] * dv)
                    return 0
                lax.fori_loop(0, RB, u_row, 0)
                pltpu.sync_copy(buf1.at[pl.ds(0, RB)],
                                u_hbm.at[pl.ds(base, RB)])
                return 0
            with jax.named_scope("p4_u"):
                lax.fori_loop(0, nu, u_chunk, 0)
            plsc.subcore_barrier()

            def gather(j, b):
                return pltpu.async_copy(u_hbm.at[sd_t.at[j, 0]], bufs[b],
                                        semg[b])

            def drain_g(b):
                pltpu.make_async_copy(u_hbm.at[sd_t.at[0, 0]], bufs[b],
                                      semg[b]).wait()

            def drain_s(b):
                pltpu.make_async_copy(bufs[b], z_sp.at[sd_t.at[0, 1]],
                                      sems[b]).wait()

            def scat(j, b):
                return pltpu.async_copy(bufs[b], z_sp.at[sd_t.at[j, 1]],
                                        sems[b], add=True)

            def step(j, b, first):
                drain_g(b)
                scat(j, b)
                if not first:
                    drain_s((b + 2) % NBUF)
                gather(j + 2, (b + 2) % NBUF)

            for h in range(2):
                pltpu.sync_copy(gsd_hbm.at[s, pl.ds(h * NCHH, NCHH)], sd_t)
                gather(0, 0)
                gather(1, 1)
                step(0, 0, True)
                step(1, 1, True)

                def quad(m, _):
                    j = 4 * m + 2
                    for t in range(NBUF):
                        step(j + t, (2 + t) % NBUF, False)
                    return 0
                lax.fori_loop(0, (NCHH - 4) // NBUF, quad, 0)
                for j in (NCHH - 2, NCHH - 1):
                    b = j % NBUF
                    drain_s((b + 2) % NBUF)
                    drain_g(b)
                    scat(j, b)
                for j in (NCHH - 2, NCHH - 1):
                    drain_s(j % NBUF)
            plsc.subcore_barrier()

            def z_chunk(kk, _):
                base = (s + kk * NS) * RB
                pltpu.async_copy(z_sp.at[pl.ds(base, RB)],
                                 z_hbm.at[pl.ds(base, RB)], semg0)
                return 0
            lax.fori_loop(0, nu, z_chunk, 0)

            def z_drain(kk, _):
                pltpu.make_async_copy(z_sp.at[pl.ds(0, RB)],
                                      z_hbm.at[pl.ds(0, RB)], semg0).wait()
                return 0
            lax.fori_loop(0, nu, z_drain, 0)

        pl.when(c == 0)(lambda: halfwork(u0_hbm, zA_hbm, 0))
        pl.when(c == 1)(lambda: halfwork(u1_hbm, zB_hbm, 1))

        pl.when(jnp.logical_and(c == 0, s == 0))(
            lambda: pltpu.sync_copy(dinv_sp, dinv_hbm))

    return body(x3, gsd)


def _tc_heads(zA, zB, u0, u1, dvcol, WnT, WpT, WpiT, bn, bp, bpi):
    ROWS = 1000
    grid = (N // ROWS,)

    def body(zA_r, zB_r, u0_r, u1_r, dv_r, WnT_r, WpT_r, WpiT_r,
             bn_r, bp_r, bpi_r, on_r, op_r, opi_r):
        dv = dv_r[...]
        A0 = (zA_r[...] + u0_r[...]) * dv
        A1 = (zB_r[...] + u1_r[...]) * dv

        def head(WT_r, b_r):
            WT = WT_r[...]
            acc = jnp.dot(A0, WT[:CH, :], preferred_element_type=jnp.float32)
            acc = acc + jnp.dot(A1, WT[CH:, :],
                                preferred_element_type=jnp.float32)
            return acc + b_r[...]

        a = head(WnT_r, bn_r)
        on_r[...] = jnp.maximum(a, 0.0) + jnp.log(1.0 + jnp.exp(-jnp.abs(a)))
        p = head(WpT_r, bp_r)
        op_r[...] = 1.0 / (1.0 + jnp.exp(-p))
        q = head(WpiT_r, bpi_r)
        opi_r[...] = 1.0 / (1.0 + jnp.exp(-q))

    row_spec = lambda cols: pl.BlockSpec((ROWS, cols), lambda i: (i, 0))
    full_spec = lambda r, cols: pl.BlockSpec((r, cols), lambda i: (0, 0))
    return pl.pallas_call(
        body,
        grid=grid,
        in_specs=[
            row_spec(CH), row_spec(CH), row_spec(CH), row_spec(CH),
            row_spec(1),
            full_spec(CIN, CIN), full_spec(CIN, CIN), full_spec(CIN, CIN),
            full_spec(1, CIN), full_spec(1, CIN), full_spec(1, CIN),
        ],
        out_specs=[row_spec(CIN), row_spec(CIN), row_spec(CIN)],
        out_shape=[jax.ShapeDtypeStruct((N, CIN), jnp.float32)] * 3,
    )(zA, zB, u0, u1, dvcol, WnT, WpT, WpiT, bn, bp, bpi)


def kernel(x, edge_index, Wn, bn, Wp, bp, Wpi, bpi):
    src = edge_index[0].astype(jnp.int32)
    dst = edge_index[1].astype(jnp.int32)

    npad = EP - ET
    pad_src = jnp.broadcast_to(jnp.arange(npad, dtype=jnp.int32) % N,
                               (NS, npad))
    pad_dst = jnp.broadcast_to(
        N + (jnp.arange(npad, dtype=jnp.int32) % PAD_SPREAD), (NS, npad))
    gsrc = jnp.concatenate([src.reshape(NS, ET), pad_src], axis=1)
    gdst = jnp.concatenate([dst.reshape(NS, ET), pad_dst], axis=1)
    gsd = jnp.stack([gsrc.reshape(NS, NCH, K), gdst.reshape(NS, NCH, K)],
                    axis=2)

    zA, zB, dinv, u0, u1 = _sc_aggregate(x.reshape(N, 2, CH), gsd)
    dvcol = dinv[:N].reshape(N, 1)

    out = _tc_heads(zA, zB, u0, u1, dvcol,
                    Wn.T, Wp.T, Wpi.T,
                    bn.reshape(1, CIN), bp.reshape(1, CIN),
                    bpi.reshape(1, CIN))
    return tuple(out)


# speedup vs baseline: 1.0430x; 1.0430x over previous
import functools

import jax
import jax.numpy as jnp
from jax import lax
from jax.experimental import pallas as pl
from jax.experimental.pallas import tpu as pltpu
from jax.experimental.pallas import tpu_sc as plsc

N = 10000
CIN = 256
CH = 128
E = 160000
NS = 16
NC = 2
ET = E // NS
K = 64
NCH = 160
NCHH = NCH // 2
NBUF = 4
EP = NCH * K
PAD_SPREAD = 128
H = 10240
SLICE = H // NS
L = 16
RB = 40
NRCH = N // RB


def _rsqrt_newton(d):
    i = plsc.bitcast(d, jnp.int32)
    y = plsc.bitcast(jnp.int32(0x5F3759DF) - (i >> 1), jnp.float32)
    for _ in range(3):
        y = y * (1.5 - 0.5 * d * y * y)
    return y


def _sc_aggregate(x3, gsd):
    mesh = plsc.VectorSubcoreMesh(core_axis_name="c", subcore_axis_name="s")

    out_type = [
        jax.ShapeDtypeStruct((N, CH), jnp.float32),
        jax.ShapeDtypeStruct((N, CH), jnp.float32),
        jax.ShapeDtypeStruct((H,), jnp.float32),
        jax.ShapeDtypeStruct((N, CH), jnp.float32),
        jax.ShapeDtypeStruct((N, CH), jnp.float32),
    ]
    scratch = [
        pltpu.VMEM((NCHH, 2, K), jnp.int32),
        pltpu.VMEM((K,), jnp.float32),
        pltpu.VMEM((704,), jnp.float32),
        pltpu.VMEM((K,), jnp.float32),
        pltpu.VMEM((K, CH), jnp.float32),
        pltpu.VMEM((K, CH), jnp.float32),
        pltpu.VMEM((K, CH), jnp.float32),
        pltpu.VMEM((K, CH), jnp.float32),
        pltpu.VMEM_SHARED((H, CH), jnp.float32),
        pltpu.VMEM_SHARED((H,), jnp.float32),
        pltpu.VMEM_SHARED((H,), jnp.float32),
    ] + [pltpu.SemaphoreType.DMA] * (2 * NBUF)

    @functools.partial(
        pl.kernel, out_type=out_type, mesh=mesh, scratch_types=scratch,
        compiler_params=pltpu.CompilerParams(use_tc_tiling_on_sc=False,
                                             needs_layout_passes=False))
    def body(x3_hbm, gsd_hbm, zA_hbm, zB_hbm, dinv_hbm, u0_hbm, u1_hbm,
             sd_t, ones_t, sl_t, dv_t, buf0, buf1, buf2, buf3,
             z_sp, hist_sp, dinv_sp,
             semg0, semg1, semg2, semg3, sems0, sems1, sems2, sems3):
        bufs = (buf0, buf1, buf2, buf3)
        semg = (semg0, semg1, semg2, semg3)
        sems = (sems0, sems1, sems2, sems3)
        c = lax.axis_index("c")
        s = lax.axis_index("s")
        zeros16 = jnp.zeros((L,), jnp.float32)
        ones16 = jnp.ones((L,), jnp.float32)

        def zero_sl(i, _):
            sl_t[pl.ds(i * L, L)] = zeros16
            return 0
        lax.fori_loop(0, SLICE // L, zero_sl, 0, unroll=4)

        def zero_buf0(i, _):
            for k in range(CH // L):
                buf0[i, pl.ds(k * L, L)] = zeros16
            return 0
        lax.fori_loop(0, K, zero_buf0, 0, unroll=2)

        for k in range(K // L):
            ones_t[pl.ds(k * L, L)] = ones16

        pltpu.sync_copy(sl_t.at[pl.ds(0, SLICE)],
                        hist_sp.at[pl.ds(s * SLICE, SLICE)])
        plsc.subcore_barrier()

        _p1 = jax.named_scope("p1_degree"); _p1.__enter__()
        D = 16

        def hfire(j):
            pltpu.async_copy(ones_t, hist_sp.at[sd_t.at[j, 1]], semg0,
                             add=True)

        def hdrain(j, _):
            pltpu.make_async_copy(ones_t, hist_sp.at[sd_t.at[0, 1]], semg0
                                  ).wait()
            return 0

        for h in range(2):
            pltpu.sync_copy(gsd_hbm.at[s, pl.ds(h * NCHH, NCHH)], sd_t)
            for j in range(D):
                hfire(j)

            def hist_step(j, _):
                hfire(j + D)
                hdrain(j, None)
                return 0
            lax.fori_loop(0, NCHH - D, hist_step, 0)
            lax.fori_loop(0, D, hdrain, 0)
        plsc.subcore_barrier()

        _p1.__exit__(None, None, None)
        _p2 = jax.named_scope("p2_dinv_zero"); _p2.__enter__()
        pltpu.sync_copy(hist_sp.at[pl.ds(s * SLICE, SLICE)],
                        sl_t.at[pl.ds(0, SLICE)])

        def dinv_step(m, _):
            d = sl_t[pl.ds(m * L, L)] + 1.0
            sl_t[pl.ds(m * L, L)] = _rsqrt_newton(d)
            return 0
        lax.fori_loop(0, SLICE // L, dinv_step, 0)
        pltpu.sync_copy(sl_t.at[pl.ds(0, SLICE)],
                        dinv_sp.at[pl.ds(s * SLICE, SLICE)])

        for m in range(SLICE // K):
            pltpu.sync_copy(buf0, z_sp.at[pl.ds(s * SLICE + m * K, K)])
        plsc.subcore_barrier()

        _p2.__exit__(None, None, None)
        nu = jnp.where(s < NRCH - (NRCH // NS) * NS, NRCH // NS + 1,
                       NRCH // NS)

        def halfwork(u_hbm, z_hbm, hw_c):
            NUC = 16

            def u_base(kk):
                return jnp.minimum(s + kk * NS, NRCH - 1) * RB

            def u_load(kk, b):
                pltpu.async_copy(x3_hbm.at[pl.ds(u_base(kk), RB), hw_c],
                                 bufs[b].at[pl.ds(0, RB)], semg[b])

            def u_drain_load(b):
                pltpu.make_async_copy(x3_hbm.at[pl.ds(0, RB), 0],
                                      bufs[b].at[pl.ds(0, RB)],
                                      semg[b]).wait()

            def u_drain_store(b):
                pltpu.make_async_copy(bufs[b].at[pl.ds(0, RB)],
                                      u_hbm.at[pl.ds(0, RB)], sems[b]).wait()

            for k in range(NUC):
                pltpu.async_copy(dinv_sp.at[pl.ds(u_base(k), RB)],
                                 sl_t.at[pl.ds(RB * k, RB)], semg[0])
            for k in range(NUC):
                pltpu.make_async_copy(dinv_sp.at[pl.ds(0, RB)],
                                      sl_t.at[pl.ds(RB * k, RB)],
                                      semg[0]).wait()

            def u_body(kk, t, issue_next, drain_next):
                if issue_next:
                    nb = (t + 1) % NBUF
                    if drain_next:
                        u_drain_store(nb)
                    u_load(kk + 1, nb)
                u_drain_load(t)

                def u_row(i, _):
                    dv = sl_t[pl.ds(RB * kk + i, L)][0]
                    for k in range(CH // L):
                        bufs[t][i, pl.ds(k * L, L)] = (
                            bufs[t][i, pl.ds(k * L, L)] * dv)
                    return 0
                lax.fori_loop(0, RB, u_row, 0)
                pltpu.async_copy(bufs[t].at[pl.ds(0, RB)],
                                 u_hbm.at[pl.ds(u_base(kk), RB)], sems[t])

            u_load(0, 0)
            for t in range(NBUF):
                u_body(t, t, True, t == NBUF - 1)

            def u_quad(m, _):
                for t in range(NBUF):
                    u_body(NBUF * m + t, t, True, True)
                return 0
            lax.fori_loop(1, NUC // NBUF - 1, u_quad, 0)
            for t in range(NBUF):
                u_body(NUC - NBUF + t, t, t != NBUF - 1, t != NBUF - 1)
            for t in range(NBUF):
                u_drain_store(t)
            plsc.subcore_barrier()

            def gather(j, b):
                return pltpu.async_copy(u_hbm.at[sd_t.at[j, 0]], bufs[b],
                                        semg[b])

            def drain_g(b):
                pltpu.make_async_copy(u_hbm.at[sd_t.at[0, 0]], bufs[b],
                                      semg[b]).wait()

            def drain_s(b):
                pltpu.make_async_copy(bufs[b], z_sp.at[sd_t.at[0, 1]],
                                      sems[b]).wait()

            def scat(j, b):
                return pltpu.async_copy(bufs[b], z_sp.at[sd_t.at[j, 1]],
                                        sems[b], add=True)

            def step(j, b, first):
                drain_g(b)
                scat(j, b)
                if not first:
                    drain_s((b + 2) % NBUF)
                gather(j + 2, (b + 2) % NBUF)

            for h in range(2):
                pltpu.sync_copy(gsd_hbm.at[s, pl.ds(h * NCHH, NCHH)], sd_t)
                gather(0, 0)
                gather(1, 1)
                step(0, 0, True)
                step(1, 1, True)

                def quad(m, _):
                    j = 4 * m + 2
                    for t in range(NBUF):
                        step(j + t, (2 + t) % NBUF, False)
                    return 0
                lax.fori_loop(0, (NCHH - 4) // NBUF, quad, 0)
                for j in (NCHH - 2, NCHH - 1):
                    b = j % NBUF
                    drain_s((b + 2) % NBUF)
                    drain_g(b)
                    scat(j, b)
                for j in (NCHH - 2, NCHH - 1):
                    drain_s(j % NBUF)
            plsc.subcore_barrier()

            def z_chunk(kk, _):
                base = (s + kk * NS) * RB
                pltpu.async_copy(z_sp.at[pl.ds(base, RB)],
                                 z_hbm.at[pl.ds(base, RB)], semg0)
                return 0
            lax.fori_loop(0, nu, z_chunk, 0)

            def z_drain(kk, _):
                pltpu.make_async_copy(z_sp.at[pl.ds(0, RB)],
                                      z_hbm.at[pl.ds(0, RB)], semg0).wait()
                return 0
            lax.fori_loop(0, nu, z_drain, 0)

        pl.when(c == 0)(lambda: halfwork(u0_hbm, zA_hbm, 0))
        pl.when(c == 1)(lambda: halfwork(u1_hbm, zB_hbm, 1))

        pl.when(jnp.logical_and(c == 0, s == 0))(
            lambda: pltpu.sync_copy(dinv_sp, dinv_hbm))

    return body(x3, gsd)


def _tc_heads(zA, zB, u0, u1, dvcol, WnT, WpT, WpiT, bn, bp, bpi):
    ROWS = 1000
    grid = (N // ROWS,)

    def body(zA_r, zB_r, u0_r, u1_r, dv_r, WnT_r, WpT_r, WpiT_r,
             bn_r, bp_r, bpi_r, on_r, op_r, opi_r):
        dv = dv_r[...]
        A0 = (zA_r[...] + u0_r[...]) * dv
        A1 = (zB_r[...] + u1_r[...]) * dv

        def head(WT_r, b_r):
            WT = WT_r[...]
            acc = jnp.dot(A0, WT[:CH, :], preferred_element_type=jnp.float32)
            acc = acc + jnp.dot(A1, WT[CH:, :],
                                preferred_element_type=jnp.float32)
            return acc + b_r[...]

        a = head(WnT_r, bn_r)
        on_r[...] = jnp.maximum(a, 0.0) + jnp.log(1.0 + jnp.exp(-jnp.abs(a)))
        p = head(WpT_r, bp_r)
        op_r[...] = 1.0 / (1.0 + jnp.exp(-p))
        q = head(WpiT_r, bpi_r)
        opi_r[...] = 1.0 / (1.0 + jnp.exp(-q))

    row_spec = lambda cols: pl.BlockSpec((ROWS, cols), lambda i: (i, 0))
    full_spec = lambda r, cols: pl.BlockSpec((r, cols), lambda i: (0, 0))
    return pl.pallas_call(
        body,
        grid=grid,
        in_specs=[
            row_spec(CH), row_spec(CH), row_spec(CH), row_spec(CH),
            row_spec(1),
            full_spec(CIN, CIN), full_spec(CIN, CIN), full_spec(CIN, CIN),
            full_spec(1, CIN), full_spec(1, CIN), full_spec(1, CIN),
        ],
        out_specs=[row_spec(CIN), row_spec(CIN), row_spec(CIN)],
        out_shape=[jax.ShapeDtypeStruct((N, CIN), jnp.float32)] * 3,
    )(zA, zB, u0, u1, dvcol, WnT, WpT, WpiT, bn, bp, bpi)


def kernel(x, edge_index, Wn, bn, Wp, bp, Wpi, bpi):
    src = edge_index[0].astype(jnp.int32)
    dst = edge_index[1].astype(jnp.int32)

    npad = EP - ET
    pad_src = jnp.broadcast_to(jnp.arange(npad, dtype=jnp.int32) % N,
                               (NS, npad))
    pad_dst = jnp.broadcast_to(
        N + (jnp.arange(npad, dtype=jnp.int32) % PAD_SPREAD), (NS, npad))
    gsrc = jnp.concatenate([src.reshape(NS, ET), pad_src], axis=1)
    gdst = jnp.concatenate([dst.reshape(NS, ET), pad_dst], axis=1)
    gsd = jnp.stack([gsrc.reshape(NS, NCH, K), gdst.reshape(NS, NCH, K)],
                    axis=2)

    zA, zB, dinv, u0, u1 = _sc_aggregate(x.reshape(N, 2, CH), gsd)
    dvcol = dinv[:N].reshape(N, 1)

    out = _tc_heads(zA, zB, u0, u1, dvcol,
                    Wn.T, Wp.T, Wpi.T,
                    bn.reshape(1, CIN), bp.reshape(1, CIN),
                    bpi.reshape(1, CIN))
    return tuple(out)


# speedup vs baseline: 1.0453x; 1.0023x over previous
import functools

import jax
import jax.numpy as jnp
from jax import lax
from jax.experimental import pallas as pl
from jax.experimental.pallas import tpu as pltpu
from jax.experimental.pallas import tpu_sc as plsc

N = 10000
CIN = 256
CH = 128
E = 160000
NS = 16
NC = 2
ET = E // NS
K = 64
NCH = 160
NCHH = NCH // 2
NBUF = 4
EP = NCH * K
PAD_SPREAD = 128
H = 10240
SLICE = H // NS
L = 16
RB = 40
NRCH = N // RB


def _rsqrt_newton(d):
    i = plsc.bitcast(d, jnp.int32)
    y = plsc.bitcast(jnp.int32(0x5F3759DF) - (i >> 1), jnp.float32)
    for _ in range(3):
        y = y * (1.5 - 0.5 * d * y * y)
    return y


def _sc_aggregate(x3, gsd):
    mesh = plsc.VectorSubcoreMesh(core_axis_name="c", subcore_axis_name="s")

    out_type = [
        jax.ShapeDtypeStruct((N, CH), jnp.float32),
        jax.ShapeDtypeStruct((N, CH), jnp.float32),
        jax.ShapeDtypeStruct((H,), jnp.float32),
        jax.ShapeDtypeStruct((N, CH), jnp.float32),
        jax.ShapeDtypeStruct((N, CH), jnp.float32),
    ]
    scratch = [
        pltpu.VMEM((NCHH, 2, K), jnp.int32),
        pltpu.VMEM((K,), jnp.float32),
        pltpu.VMEM((704,), jnp.float32),
        pltpu.VMEM((K,), jnp.float32),
        pltpu.VMEM((K, CH), jnp.float32),
        pltpu.VMEM((K, CH), jnp.float32),
        pltpu.VMEM((K, CH), jnp.float32),
        pltpu.VMEM((K, CH), jnp.float32),
        pltpu.VMEM_SHARED((H, CH), jnp.float32),
        pltpu.VMEM_SHARED((H,), jnp.float32),
        pltpu.VMEM_SHARED((H,), jnp.float32),
    ] + [pltpu.SemaphoreType.DMA] * (2 * NBUF)

    @functools.partial(
        pl.kernel, out_type=out_type, mesh=mesh, scratch_types=scratch,
        compiler_params=pltpu.CompilerParams(use_tc_tiling_on_sc=False,
                                             needs_layout_passes=False))
    def body(x3_hbm, gsd_hbm, zA_hbm, zB_hbm, dinv_hbm, u0_hbm, u1_hbm,
             sd_t, ones_t, sl_t, dv_t, buf0, buf1, buf2, buf3,
             z_sp, hist_sp, dinv_sp,
             semg0, semg1, semg2, semg3, sems0, sems1, sems2, sems3):
        bufs = (buf0, buf1, buf2, buf3)
        semg = (semg0, semg1, semg2, semg3)
        sems = (sems0, sems1, sems2, sems3)
        c = lax.axis_index("c")
        s = lax.axis_index("s")
        zeros16 = jnp.zeros((L,), jnp.float32)
        ones16 = jnp.ones((L,), jnp.float32)

        def zero_sl(i, _):
            sl_t[pl.ds(i * L, L)] = zeros16
            return 0
        lax.fori_loop(0, SLICE // L, zero_sl, 0, unroll=4)

        def zero_buf0(i, _):
            for k in range(CH // L):
                buf0[i, pl.ds(k * L, L)] = zeros16
            return 0
        lax.fori_loop(0, K, zero_buf0, 0, unroll=2)

        for k in range(K // L):
            ones_t[pl.ds(k * L, L)] = ones16

        pltpu.sync_copy(sl_t.at[pl.ds(0, SLICE)],
                        hist_sp.at[pl.ds(s * SLICE, SLICE)])
        plsc.subcore_barrier()

        _p1 = jax.named_scope("p1_degree"); _p1.__enter__()
        D = 16

        def hfire(j):
            pltpu.async_copy(ones_t, hist_sp.at[sd_t.at[j, 1]], semg0,
                             add=True)

        def hdrain(j, _):
            pltpu.make_async_copy(ones_t, hist_sp.at[sd_t.at[0, 1]], semg0
                                  ).wait()
            return 0

        for h in range(2):
            pltpu.sync_copy(gsd_hbm.at[s, pl.ds(h * NCHH, NCHH)], sd_t)
            for j in range(D):
                hfire(j)

            def hist_step(j, _):
                hfire(j + D)
                hdrain(j, None)
                return 0
            lax.fori_loop(0, NCHH - D, hist_step, 0)
            lax.fori_loop(0, D, hdrain, 0)
        plsc.subcore_barrier()

        _p1.__exit__(None, None, None)
        _p2 = jax.named_scope("p2_dinv_zero"); _p2.__enter__()
        pltpu.sync_copy(hist_sp.at[pl.ds(s * SLICE, SLICE)],
                        sl_t.at[pl.ds(0, SLICE)])

        def dinv_step(m, _):
            d = sl_t[pl.ds(m * L, L)] + 1.0
            sl_t[pl.ds(m * L, L)] = _rsqrt_newton(d)
            return 0
        lax.fori_loop(0, SLICE // L, dinv_step, 0)
        pltpu.sync_copy(sl_t.at[pl.ds(0, SLICE)],
                        dinv_sp.at[pl.ds(s * SLICE, SLICE)])

        for m in range(SLICE // K):
            pltpu.sync_copy(buf0, z_sp.at[pl.ds(s * SLICE + m * K, K)])
        plsc.subcore_barrier()

        _p2.__exit__(None, None, None)
        nu = jnp.where(s < NRCH - (NRCH // NS) * NS, NRCH // NS + 1,
                       NRCH // NS)

        def halfwork(u_hbm, z_hbm, hw_c):
            NUC = 16

            def u_base(kk):
                return jnp.minimum(s + kk * NS, NRCH - 1) * RB

            def u_load(kk, b):
                pltpu.async_copy(x3_hbm.at[pl.ds(u_base(kk), RB), hw_c],
                                 bufs[b].at[pl.ds(0, RB)], semg[b])

            def u_drain_load(b):
                pltpu.make_async_copy(x3_hbm.at[pl.ds(0, RB), 0],
                                      bufs[b].at[pl.ds(0, RB)],
                                      semg[b]).wait()

            def u_drain_store(b):
                pltpu.make_async_copy(bufs[b].at[pl.ds(0, RB)],
                                      u_hbm.at[pl.ds(0, RB)], sems[b]).wait()

            for k in range(NUC):
                pltpu.async_copy(dinv_sp.at[pl.ds(u_base(k), RB)],
                                 sl_t.at[pl.ds(RB * k, RB)], semg[0])
            for k in range(NUC):
                pltpu.make_async_copy(dinv_sp.at[pl.ds(0, RB)],
                                      sl_t.at[pl.ds(RB * k, RB)],
                                      semg[0]).wait()

            def u_body(kk, t, issue_next, drain_next):
                if issue_next:
                    nb = (t + 1) % NBUF
                    if drain_next:
                        u_drain_store(nb)
                    u_load(kk + 1, nb)
                u_drain_load(t)

                def u_row(i, _):
                    dv = sl_t[pl.ds(RB * kk + i, L)][0]
                    for k in range(CH // L):
                        bufs[t][i, pl.ds(k * L, L)] = (
                            bufs[t][i, pl.ds(k * L, L)] * dv)
                    return 0
                lax.fori_loop(0, RB, u_row, 0)
                pltpu.async_copy(bufs[t].at[pl.ds(0, RB)],
                                 u_hbm.at[pl.ds(u_base(kk), RB)], sems[t])

            u_load(0, 0)
            for t in range(NBUF):
                u_body(t, t, True, t == NBUF - 1)

            def u_quad(m, _):
                for t in range(NBUF):
                    u_body(NBUF * m + t, t, True, True)
                return 0
            lax.fori_loop(1, NUC // NBUF - 1, u_quad, 0)
            for t in range(NBUF):
                u_body(NUC - NBUF + t, t, t != NBUF - 1, t != NBUF - 1)
            for t in range(NBUF):
                u_drain_store(t)
            plsc.subcore_barrier()

            def gather(j, b):
                return pltpu.async_copy(u_hbm.at[sd_t.at[j, 0]], bufs[b],
                                        semg[b])

            def drain_g(b):
                pltpu.make_async_copy(u_hbm.at[sd_t.at[0, 0]], bufs[b],
                                      semg[b]).wait()

            def drain_s(b):
                pltpu.make_async_copy(bufs[b], z_sp.at[sd_t.at[0, 1]],
                                      sems[b]).wait()

            def scat(j, b):
                return pltpu.async_copy(bufs[b], z_sp.at[sd_t.at[j, 1]],
                                        sems[b], add=True)

            def step(j, b, first):
                drain_g(b)
                scat(j, b)
                if not first:
                    drain_s((b + 2) % NBUF)
                gather(j + 2, (b + 2) % NBUF)

            for h in range(2):
                pltpu.sync_copy(gsd_hbm.at[s, pl.ds(h * NCHH, NCHH)], sd_t)
                gather(0, 0)
                gather(1, 1)
                step(0, 0, True)
                step(1, 1, True)

                def quad(m, _):
                    j = 4 * m + 2
                    for t in range(NBUF):
                        step(j + t, (2 + t) % NBUF, False)
                    return 0
                lax.fori_loop(0, (NCHH - 4) // NBUF, quad, 0)
                for j in (NCHH - 2, NCHH - 1):
                    b = j % NBUF
                    drain_s((b + 2) % NBUF)
                    drain_g(b)
                    scat(j, b)
                for j in (NCHH - 2, NCHH - 1):
                    drain_s(j % NBUF)
            plsc.subcore_barrier()

            def z_chunk(kk, _):
                base = (s + kk * NS) * RB
                pltpu.async_copy(z_sp.at[pl.ds(base, RB)],
                                 z_hbm.at[pl.ds(base, RB)], semg0)
                return 0
            lax.fori_loop(0, nu, z_chunk, 0)

            def z_drain(kk, _):
                pltpu.make_async_copy(z_sp.at[pl.ds(0, RB)],
                                      z_hbm.at[pl.ds(0, RB)], semg0).wait()
                return 0
            lax.fori_loop(0, nu, z_drain, 0)

        pl.when(c == 0)(lambda: halfwork(u0_hbm, zA_hbm, 0))
        pl.when(c == 1)(lambda: halfwork(u1_hbm, zB_hbm, 1))

        pl.when(jnp.logical_and(c == 0, s == 0))(
            lambda: pltpu.sync_copy(dinv_sp, dinv_hbm))

    return body(x3, gsd)


def _tc_heads(zA, zB, u0, u1, dvcol, WnT, WpT, WpiT, bn, bp, bpi):
    ROWS = 2000
    grid = (N // ROWS,)

    def body(zA_r, zB_r, u0_r, u1_r, dv_r, WnT_r, WpT_r, WpiT_r,
             bn_r, bp_r, bpi_r, on_r, op_r, opi_r):
        dv = dv_r[...]
        A0 = (zA_r[...] + u0_r[...]) * dv
        A1 = (zB_r[...] + u1_r[...]) * dv

        def head(WT_r, b_r):
            WT = WT_r[...]
            acc = jnp.dot(A0, WT[:CH, :], preferred_element_type=jnp.float32)
            acc = acc + jnp.dot(A1, WT[CH:, :],
                                preferred_element_type=jnp.float32)
            return acc + b_r[...]

        a = head(WnT_r, bn_r)
        on_r[...] = jnp.maximum(a, 0.0) + jnp.log(1.0 + jnp.exp(-jnp.abs(a)))
        p = head(WpT_r, bp_r)
        op_r[...] = 1.0 / (1.0 + jnp.exp(-p))
        q = head(WpiT_r, bpi_r)
        opi_r[...] = 1.0 / (1.0 + jnp.exp(-q))

    row_spec = lambda cols: pl.BlockSpec((ROWS, cols), lambda i: (i, 0))
    full_spec = lambda r, cols: pl.BlockSpec((r, cols), lambda i: (0, 0))
    return pl.pallas_call(
        body,
        grid=grid,
        in_specs=[
            row_spec(CH), row_spec(CH), row_spec(CH), row_spec(CH),
            row_spec(1),
            full_spec(CIN, CIN), full_spec(CIN, CIN), full_spec(CIN, CIN),
            full_spec(1, CIN), full_spec(1, CIN), full_spec(1, CIN),
        ],
        out_specs=[row_spec(CIN), row_spec(CIN), row_spec(CIN)],
        out_shape=[jax.ShapeDtypeStruct((N, CIN), jnp.float32)] * 3,
    )(zA, zB, u0, u1, dvcol, WnT, WpT, WpiT, bn, bp, bpi)


def kernel(x, edge_index, Wn, bn, Wp, bp, Wpi, bpi):
    src = edge_index[0].astype(jnp.int32)
    dst = edge_index[1].astype(jnp.int32)

    npad = EP - ET
    pad_src = jnp.broadcast_to(jnp.arange(npad, dtype=jnp.int32) % N,
                               (NS, npad))
    pad_dst = jnp.broadcast_to(
        N + (jnp.arange(npad, dtype=jnp.int32) % PAD_SPREAD), (NS, npad))
    gsrc = jnp.concatenate([src.reshape(NS, ET), pad_src], axis=1)
    gdst = jnp.concatenate([dst.reshape(NS, ET), pad_dst], axis=1)
    gsd = jnp.stack([gsrc.reshape(NS, NCH, K), gdst.reshape(NS, NCH, K)],
                    axis=2)

    zA, zB, dinv, u0, u1 = _sc_aggregate(x.reshape(N, 2, CH), gsd)
    dvcol = dinv[:N].reshape(N, 1)

    out = _tc_heads(zA, zB, u0, u1, dvcol,
                    Wn.T, Wp.T, Wpi.T,
                    bn.reshape(1, CIN), bp.reshape(1, CIN),
                    bpi.reshape(1, CIN))
    return tuple(out)
